# baseline (speedup 1.0000x reference)
import functools

import jax
import jax.numpy as jnp
from jax import lax
from jax.experimental import pallas as pl
from jax.experimental.pallas import tpu as pltpu
from jax.experimental.pallas import tpu_sc as plsc

_LEN_PENALTY_RATIO = 0.8
_ROW_BLOCK = 8
_BIG = 2**30


def _argmax4(y, vocab):
    ns = 4
    b0 = ((vocab // ns) // 128) * 128
    bounds = [i * b0 for i in range(ns)] + [vocab]
    vs, as_ = [], []
    for st, en in zip(bounds[:-1], bounds[1:]):
        xs = y[:, st:en]
        vs.append(jnp.max(xs, axis=1, keepdims=True))
        as_.append(jnp.argmax(xs, axis=1, keepdims=True).astype(jnp.int32) + st)
    v, a = vs[0], as_[0]
    for i in range(1, ns):
        better = vs[i] > v
        a = jnp.where(better, as_[i], a)
        v = jnp.where(better, vs[i], v)
    return v, a


def _rowtopk_body(k, rb, vocab, logits_ref, cur_ref, size_ref, eos_ref,
                  vals_ref, idxs_ref, stats_ref, xs_ref):
    x = logits_ref[...]
    v, a = _argmax4(x, vocab)
    s = jnp.sum(jnp.exp(x - v), axis=1, keepdims=True)
    cur = cur_ref[...]
    pen = jnp.power((size_ref[...] + 6.0) / 6.0, _LEN_PENALTY_RATIO)
    adj = cur - v - jnp.log(s)
    stats_ref[...] = jnp.concatenate([adj, pen, cur, eos_ref[...]], axis=1)

    xs_ref[...] = x
    lane = lax.broadcasted_iota(jnp.int32, (1, 128), 1)
    vals, idxs = [v], [a]
    for _ in range(1, k):
        for r in range(rb):
            ar = a[r, 0]
            base = pl.multiple_of((ar // 128) * 128, 128)
            off = ar - base
            chunk = xs_ref[pl.ds(r, 1), pl.ds(base, 128)]
            xs_ref[pl.ds(r, 1), pl.ds(base, 128)] = jnp.where(
                lane == off, -jnp.inf, chunk)
        v, a = _argmax4(xs_ref[...], vocab)
        vals.append(v)
        idxs.append(a)
    vals_ref[...] = jnp.concatenate(vals, axis=1)
    idxs_ref[...] = jnp.concatenate(idxs, axis=1)


def _sc_merge_body(k, vocab, vals_hbm, idxs_hbm, stats_hbm, tbo_hbm,
                   tops_hbm, words_hbm, rows_hbm,
                   vals_v, idxs_v, stats_v, rows_v, gidx_v,
                   tops_v, words_v, sem):
    wid = lax.axis_index("s") * 2 + lax.axis_index("c")
    pltpu.sync_copy(vals_hbm.at[pl.ds(wid * 8, 8)], vals_v)
    pltpu.sync_copy(idxs_hbm.at[pl.ds(wid * 8, 8)], idxs_v)
    pltpu.sync_copy(stats_hbm.at[pl.ds(wid * 8, 8)], stats_v)
    lane = lax.broadcasted_iota(jnp.int32, (16,), 0)
    neg = jnp.full((16,), -jnp.inf, jnp.float32)
    for bi in range(2):
        bvec = jnp.full((16,), bi, jnp.int32)
        sc_chunks, fl_chunks = [], []
        for c in range(2):
            slot = lane + 16 * c
            valid = slot < (k * k)
            beam = jnp.zeros((16,), jnp.int32)
            for g in range(1, 32 // k + 1):
                beam = beam + jnp.where(slot >= g * k, 1, 0)
            idx = idxs_v[bi, pl.ds(16 * c, 16)]
            val = vals_v[bi, pl.ds(16 * c, 16)]
            adj = plsc.load_gather(stats_v, [bvec, beam])
            pen = plsc.load_gather(stats_v, [bvec, beam + 8])
            cur = plsc.load_gather(stats_v, [bvec, beam + 16])
            eos = plsc.load_gather(stats_v, [bvec, beam + 24])
            word = jnp.where(eos > 0.5, slot - beam * k, idx)
            s = jnp.where(eos > 0.5, cur, val + adj) / pen
            sc_chunks.append(jnp.where(valid, s, neg))
            fl_chunks.append(jnp.where(valid, beam * vocab + word, _BIG))
        tops_vec = jnp.zeros((16,), jnp.float32)
        flat_vec = jnp.zeros((16,), jnp.int32)
        for t in range(k):
            m = jnp.maximum(jnp.max(sc_chunks[0]), jnp.max(sc_chunks[1]))
            fc0 = jnp.where(sc_chunks[0] == m, fl_chunks[0], _BIG)
            fc1 = jnp.where(sc_chunks[1] == m, fl_chunks[1], _BIG)
            am = jnp.minimum(jnp.min(fc0), jnp.min(fc1))
            sc_chunks[0] = jnp.where(fc0 == am, neg, sc_chunks[0])
            sc_chunks[1] = jnp.where(fc1 == am, neg, sc_chunks[1])
            tops_vec = jnp.where(lane == t, m, tops_vec)
            flat_vec = jnp.where(lane == t, am, flat_vec)
        beam_vec = jnp.zeros((16,), jnp.int32)
        for g in range(1, k):
            beam_vec = beam_vec + jnp.where(flat_vec >= g * vocab, 1, 0)
        word_vec = flat_vec - beam_vec * vocab
        b_global = wid * 2 + bi
        gidx_v[...] = jnp.where(lane < k, b_global * k + beam_vec, 0)
        pltpu.async_copy(tbo_hbm.at[gidx_v], rows_v, sem).wait()
        pltpu.sync_copy(rows_v.at[pl.ds(0, 8)], rows_hbm.at[pl.ds(b_global * 8, 8)])
        tops_v[bi, pl.ds(0, 16)] = tops_vec
        words_v[bi, pl.ds(0, 16)] = word_vec
    pltpu.sync_copy(tops_v, tops_hbm.at[pl.ds(wid * 8, 8)])
    pltpu.sync_copy(words_v, words_hbm.at[pl.ds(wid * 8, 8)])


def kernel(logits, cur_scores, cur_size, eos_mask, top_beam_outputs, beam_width):
    Bb, k, seq_len = top_beam_outputs.shape
    rows, vocab = logits.shape
    rb = _ROW_BLOCK
    grid = rows // rb

    cur2 = cur_scores.reshape(rows, 1)
    size_f = cur_size.astype(jnp.float32).reshape(rows, 1)
    eos_f = eos_mask.astype(jnp.float32).reshape(rows, 1)

    vals, idxs, stats = pl.pallas_call(
        functools.partial(_rowtopk_body, k, rb, vocab),
        grid=(grid,),
        scratch_shapes=[pltpu.VMEM((rb, vocab), jnp.float32)],
        in_specs=[
            pl.BlockSpec((rb, vocab), lambda i: (i, 0)),
            pl.BlockSpec((rb, 1), lambda i: (i, 0)),
            pl.BlockSpec((rb, 1), lambda i: (i, 0)),
            pl.BlockSpec((rb, 1), lambda i: (i, 0)),
        ],
        out_specs=[
            pl.BlockSpec((rb, k), lambda i: (i, 0)),
            pl.BlockSpec((rb, k), lambda i: (i, 0)),
            pl.BlockSpec((rb, 4), lambda i: (i, 0)),
        ],
        out_shape=[
            jax.ShapeDtypeStruct((rows, k), jnp.float32),
            jax.ShapeDtypeStruct((rows, k), jnp.int32),
            jax.ShapeDtypeStruct((rows, 4), jnp.float32),
        ],
    )(logits, cur2, size_f, eos_f)

    nw = Bb // 2
    vals_pad = jnp.zeros((nw, 8, 32), jnp.float32).at[:, :2, : k * k].set(
        vals.reshape(nw, 2, k * k)).reshape(nw * 8, 32)
    idxs_pad = jnp.zeros((nw, 8, 32), jnp.int32).at[:, :2, : k * k].set(
        idxs.reshape(nw, 2, k * k)).reshape(nw * 8, 32)
    s4 = stats.reshape(Bb, k, 4)
    stats_row = (jnp.zeros((Bb, 32), jnp.float32)
                 .at[:, 0:k].set(s4[:, :, 0])
                 .at[:, 8:8 + k].set(s4[:, :, 1])
                 .at[:, 16:16 + k].set(s4[:, :, 2])
                 .at[:, 24:24 + k].set(s4[:, :, 3]))
    stats_pad = jnp.zeros((nw, 8, 32), jnp.float32).at[:, :2, :].set(
        stats_row.reshape(nw, 2, 32)).reshape(nw * 8, 32)
    tbo2 = top_beam_outputs.reshape(Bb * k, seq_len).astype(jnp.int32)

    mesh = plsc.VectorSubcoreMesh(core_axis_name="c", subcore_axis_name="s")
    sc_merge = functools.partial(
        pl.kernel,
        mesh=mesh,
        compiler_params=pltpu.CompilerParams(
            needs_layout_passes=False, use_tc_tiling_on_sc=False),
        out_type=[
            jax.ShapeDtypeStruct((nw * 8, 16), jnp.float32),
            jax.ShapeDtypeStruct((nw * 8, 16), jnp.int32),
            jax.ShapeDtypeStruct((Bb * 8, seq_len), jnp.int32),
        ],
        scratch_types=[
            pltpu.VMEM((8, 32), jnp.float32),
            pltpu.VMEM((8, 32), jnp.int32),
            pltpu.VMEM((8, 32), jnp.float32),
            pltpu.VMEM((16, seq_len), jnp.int32),
            pltpu.VMEM((16,), jnp.int32),
            pltpu.VMEM((8, 16), jnp.float32),
            pltpu.VMEM((8, 16), jnp.int32),
            pltpu.SemaphoreType.DMA,
        ],
    )(functools.partial(_sc_merge_body, k, vocab))
    tops_pad, words_pad, rows_out = sc_merge(vals_pad, idxs_pad, stats_pad, tbo2)

    tops = tops_pad.reshape(nw, 8, 16)[:, :2, :k].reshape(Bb, k)
    words = words_pad.reshape(nw, 8, 16)[:, :2, :k].reshape(Bb, k)
    top_scores = tops + 0.0 * beam_width
    new_outputs = jnp.concatenate(
        [rows_out.reshape(Bb, 8, seq_len)[:, :k, :],
         words.astype(top_beam_outputs.dtype)[:, :, None]],
        axis=-1)
    return top_scores, new_outputs


# speedup vs baseline: 1.0440x; 1.0440x over previous
import functools

import jax
import jax.numpy as jnp
from jax import lax
from jax.experimental import pallas as pl
from jax.experimental.pallas import tpu as pltpu
from jax.experimental.pallas import tpu_sc as plsc

_LEN_PENALTY_RATIO = 0.8
_ROW_BLOCK = 8
_BIG = 2**30


def _argmax4(y, vocab):
    ns = 4
    b0 = ((vocab // ns) // 128) * 128
    bounds = [i * b0 for i in range(ns)] + [vocab]
    vs, as_ = [], []
    for st, en in zip(bounds[:-1], bounds[1:]):
        xs = y[:, st:en]
        vs.append(jnp.max(xs, axis=1, keepdims=True))
        as_.append(jnp.argmax(xs, axis=1, keepdims=True).astype(jnp.int32) + st)
    v, a = vs[0], as_[0]
    for i in range(1, ns):
        better = vs[i] > v
        a = jnp.where(better, as_[i], a)
        v = jnp.where(better, vs[i], v)
    return v, a


def _rowtopk_body(k, rb, vocab, logits_ref, cur_ref, size_ref, eos_ref,
                  vals_ref, idxs_ref, stats_ref):
    x = logits_ref[...]
    v, a = _argmax4(x, vocab)
    s = jnp.sum(jnp.exp(x - v), axis=1, keepdims=True)
    cur = cur_ref[...]
    pen = jnp.power((size_ref[...] + 6.0) / 6.0, _LEN_PENALTY_RATIO)
    adj = cur - v - jnp.log(s)
    stats_ref[...] = jnp.concatenate(
        [adj, pen, cur, eos_ref[...], jnp.zeros((rb, 4), jnp.float32)], axis=1)

    lane = lax.broadcasted_iota(jnp.int32, (1, 128), 1)
    vals, idxs = [v], [a]
    for _ in range(1, k):
        for r in range(rb):
            ar = a[r, 0]
            base = pl.multiple_of((ar // 128) * 128, 128)
            off = ar - base
            chunk = logits_ref[pl.ds(r, 1), pl.ds(base, 128)]
            logits_ref[pl.ds(r, 1), pl.ds(base, 128)] = jnp.where(
                lane == off, -jnp.inf, chunk)
        v, a = _argmax4(logits_ref[...], vocab)
        vals.append(v)
        idxs.append(a)
    pad = jnp.zeros((rb, 8 - k), jnp.float32)
    vals_ref[...] = jnp.concatenate(vals + [pad], axis=1)
    idxs_ref[...] = jnp.concatenate(idxs + [pad.astype(jnp.int32)], axis=1)


def _sc_merge_body(k, vocab, vals_hbm, idxs_hbm, stats_hbm, tbo_hbm,
                   tops_hbm, words_hbm, rows_hbm,
                   vals_v, idxs_v, stats_v, rows_v, gidx_v,
                   tops_v, words_v, sem):
    wid = lax.axis_index("s") * 2 + lax.axis_index("c")
    pltpu.sync_copy(vals_hbm, vals_v)
    pltpu.sync_copy(idxs_hbm, idxs_v)
    pltpu.sync_copy(stats_hbm, stats_v)
    lane = lax.broadcasted_iota(jnp.int32, (16,), 0)
    neg = jnp.full((16,), -jnp.inf, jnp.float32)
    for bi in range(2):
        b_glob = wid * 2 + bi
        sc_chunks, fl_chunks = [], []
        for c in range(2):
            slot = lane + 16 * c
            valid = slot < (k * k)
            beam = jnp.zeros((16,), jnp.int32)
            for g in range(1, 32 // k + 1):
                beam = beam + jnp.where(slot >= g * k, 1, 0)
            beam = jnp.minimum(beam, k - 1)
            col = jnp.minimum(slot - beam * k, k - 1)
            rowv = b_glob * k + beam
            val = plsc.load_gather(vals_v, [rowv, col])
            idx = plsc.load_gather(idxs_v, [rowv, col])
            adj = plsc.load_gather(stats_v, [rowv, jnp.zeros((16,), jnp.int32)])
            pen = plsc.load_gather(stats_v, [rowv, jnp.full((16,), 1, jnp.int32)])
            cur = plsc.load_gather(stats_v, [rowv, jnp.full((16,), 2, jnp.int32)])
            eos = plsc.load_gather(stats_v, [rowv, jnp.full((16,), 3, jnp.int32)])
            word = jnp.where(eos > 0.5, slot - beam * k, idx)
            s = jnp.where(eos > 0.5, cur, val + adj) / pen
            sc_chunks.append(jnp.where(valid, s, neg))
            fl_chunks.append(jnp.where(valid, beam * vocab + word, _BIG))
        tops_vec = jnp.zeros((16,), jnp.float32)
        flat_vec = jnp.zeros((16,), jnp.int32)
        for t in range(k):
            m = jnp.maximum(jnp.max(sc_chunks[0]), jnp.max(sc_chunks[1]))
            fc0 = jnp.where(sc_chunks[0] == m, fl_chunks[0], _BIG)
            fc1 = jnp.where(sc_chunks[1] == m, fl_chunks[1], _BIG)
            am = jnp.minimum(jnp.min(fc0), jnp.min(fc1))
            sc_chunks[0] = jnp.where(fc0 == am, neg, sc_chunks[0])
            sc_chunks[1] = jnp.where(fc1 == am, neg, sc_chunks[1])
            tops_vec = jnp.where(lane == t, m, tops_vec)
            flat_vec = jnp.where(lane == t, am, flat_vec)
        beam_vec = jnp.zeros((16,), jnp.int32)
        for g in range(1, k):
            beam_vec = beam_vec + jnp.where(flat_vec >= g * vocab, 1, 0)
        word_vec = flat_vec - beam_vec * vocab
        gidx_v[...] = jnp.where(lane < k, b_glob * k + beam_vec, 0)
        pltpu.async_copy(tbo_hbm.at[gidx_v], rows_v, sem).wait()
        pltpu.sync_copy(rows_v.at[pl.ds(0, 8)], rows_hbm.at[pl.ds(b_glob * 8, 8)])
        tops_v[bi, pl.ds(0, 16)] = tops_vec
        words_v[bi, pl.ds(0, 16)] = word_vec
    pltpu.sync_copy(tops_v, tops_hbm.at[pl.ds(wid * 8, 8)])
    pltpu.sync_copy(words_v, words_hbm.at[pl.ds(wid * 8, 8)])


def kernel(logits, cur_scores, cur_size, eos_mask, top_beam_outputs, beam_width):
    Bb, k, seq_len = top_beam_outputs.shape
    rows, vocab = logits.shape
    rb = _ROW_BLOCK
    grid = rows // rb

    cur2 = cur_scores.reshape(rows, 1)
    size_f = cur_size.astype(jnp.float32).reshape(rows, 1)
    eos_f = eos_mask.astype(jnp.float32).reshape(rows, 1)

    vals, idxs, stats = pl.pallas_call(
        functools.partial(_rowtopk_body, k, rb, vocab),
        grid=(grid,),
        in_specs=[
            pl.BlockSpec((rb, vocab), lambda i: (i, 0)),
            pl.BlockSpec((rb, 1), lambda i: (i, 0)),
            pl.BlockSpec((rb, 1), lambda i: (i, 0)),
            pl.BlockSpec((rb, 1), lambda i: (i, 0)),
        ],
        out_specs=[
            pl.BlockSpec((rb, 8), lambda i: (i, 0)),
            pl.BlockSpec((rb, 8), lambda i: (i, 0)),
            pl.BlockSpec((rb, 8), lambda i: (i, 0)),
        ],
        out_shape=[
            jax.ShapeDtypeStruct((rows, 8), jnp.float32),
            jax.ShapeDtypeStruct((rows, 8), jnp.int32),
            jax.ShapeDtypeStruct((rows, 8), jnp.float32),
        ],
    )(logits, cur2, size_f, eos_f)

    nw = Bb // 2
    tbo2 = top_beam_outputs.reshape(Bb * k, seq_len).astype(jnp.int32)

    mesh = plsc.VectorSubcoreMesh(core_axis_name="c", subcore_axis_name="s")
    sc_merge = functools.partial(
        pl.kernel,
        mesh=mesh,
        compiler_params=pltpu.CompilerParams(
            needs_layout_passes=False, use_tc_tiling_on_sc=False),
        out_type=[
            jax.ShapeDtypeStruct((nw * 8, 16), jnp.float32),
            jax.ShapeDtypeStruct((nw * 8, 16), jnp.int32),
            jax.ShapeDtypeStruct((Bb * 8, seq_len), jnp.int32),
        ],
        scratch_types=[
            pltpu.VMEM((rows, 8), jnp.float32),
            pltpu.VMEM((rows, 8), jnp.int32),
            pltpu.VMEM((rows, 8), jnp.float32),
            pltpu.VMEM((16, seq_len), jnp.int32),
            pltpu.VMEM((16,), jnp.int32),
            pltpu.VMEM((8, 16), jnp.float32),
            pltpu.VMEM((8, 16), jnp.int32),
            pltpu.SemaphoreType.DMA,
        ],
    )(functools.partial(_sc_merge_body, k, vocab))
    tops_pad, words_pad, rows_out = sc_merge(vals, idxs, stats, tbo2)

    tops = tops_pad.reshape(nw, 8, 16)[:, :2, :k].reshape(Bb, k)
    words = words_pad.reshape(nw, 8, 16)[:, :2, :k].reshape(Bb, k)
    top_scores = tops + 0.0 * beam_width
    new_outputs = jnp.concatenate(
        [rows_out.reshape(Bb, 8, seq_len)[:, :k, :],
         words.astype(top_beam_outputs.dtype)[:, :, None]],
        axis=-1)
    return top_scores, new_outputs


# speedup vs baseline: 1.0583x; 1.0137x over previous
import functools

import jax
import jax.numpy as jnp
from jax import lax
from jax.experimental import pallas as pl
from jax.experimental.pallas import tpu as pltpu
from jax.experimental.pallas import tpu_sc as plsc

_LEN_PENALTY_RATIO = 0.8
_ROW_BLOCK = 8
_BIG = 2**30


def _argmax4(y, vocab):
    ns = 4
    b0 = ((vocab // ns) // 128) * 128
    bounds = [i * b0 for i in range(ns)] + [vocab]
    vs, as_ = [], []
    for st, en in zip(bounds[:-1], bounds[1:]):
        xs = y[:, st:en]
        vs.append(jnp.max(xs, axis=1, keepdims=True))
        as_.append(jnp.argmax(xs, axis=1, keepdims=True).astype(jnp.int32) + st)
    v, a = vs[0], as_[0]
    for i in range(1, ns):
        better = vs[i] > v
        a = jnp.where(better, as_[i], a)
        v = jnp.where(better, vs[i], v)
    return v, a


def _rowtopk_body(k, rb, vocab, logits_ref, cur_ref, size_ref, eos_ref,
                  vals_ref, idxs_ref, stats_ref):
    x = logits_ref[...]
    v, a = _argmax4(x, vocab)
    s = jnp.sum(jnp.exp(x - v), axis=1, keepdims=True)
    cur = cur_ref[...]
    pen = jnp.power((size_ref[...] + 6.0) / 6.0, _LEN_PENALTY_RATIO)
    adj = cur - v - jnp.log(s)
    stats_ref[...] = jnp.concatenate(
        [adj, pen, cur, eos_ref[...], jnp.zeros((rb, 4), jnp.float32)], axis=1)

    lane = lax.broadcasted_iota(jnp.int32, (1, 128), 1)
    vals, idxs = [v], [a]
    for _ in range(1, k):
        for r in range(rb):
            ar = a[r, 0]
            base = pl.multiple_of((ar // 128) * 128, 128)
            off = ar - base
            chunk = logits_ref[pl.ds(r, 1), pl.ds(base, 128)]
            logits_ref[pl.ds(r, 1), pl.ds(base, 128)] = jnp.where(
                lane == off, -jnp.inf, chunk)
        v, a = _argmax4(logits_ref[...], vocab)
        vals.append(v)
        idxs.append(a)
    pad = jnp.zeros((rb, 8 - k), jnp.float32)
    vals_ref[...] = jnp.concatenate(vals + [pad], axis=1)
    idxs_ref[...] = jnp.concatenate(idxs + [pad.astype(jnp.int32)], axis=1)


def _sc_merge_body(k, vocab, vals_hbm, idxs_hbm, stats_hbm, tbo_hbm,
                   tops_hbm, words_hbm, rows_hbm,
                   vals_v, idxs_v, stats_v, rows_v, gidx_v,
                   tops_v, words_v, sem):
    wid = lax.axis_index("s") * 2 + lax.axis_index("c")
    h1 = pltpu.async_copy(vals_hbm, vals_v, sem)
    h2 = pltpu.async_copy(idxs_hbm, idxs_v, sem)
    h3 = pltpu.async_copy(stats_hbm, stats_v, sem)
    h1.wait()
    h2.wait()
    h3.wait()
    lane = lax.broadcasted_iota(jnp.int32, (16,), 0)
    neg = jnp.full((16,), -jnp.inf, jnp.float32)
    for bi in range(2):
        b_glob = wid * 2 + bi
        sc_chunks, fl_chunks = [], []
        for c in range(2):
            slot = lane + 16 * c
            valid = slot < (k * k)
            beam = jnp.zeros((16,), jnp.int32)
            for g in range(1, 32 // k + 1):
                beam = beam + jnp.where(slot >= g * k, 1, 0)
            beam = jnp.minimum(beam, k - 1)
            col = jnp.minimum(slot - beam * k, k - 1)
            rowv = b_glob * k + beam
            val = plsc.load_gather(vals_v, [rowv, col])
            idx = plsc.load_gather(idxs_v, [rowv, col])
            adj = plsc.load_gather(stats_v, [rowv, jnp.zeros((16,), jnp.int32)])
            pen = plsc.load_gather(stats_v, [rowv, jnp.full((16,), 1, jnp.int32)])
            cur = plsc.load_gather(stats_v, [rowv, jnp.full((16,), 2, jnp.int32)])
            eos = plsc.load_gather(stats_v, [rowv, jnp.full((16,), 3, jnp.int32)])
            word = jnp.where(eos > 0.5, slot - beam * k, idx)
            s = jnp.where(eos > 0.5, cur, val + adj) / pen
            sc_chunks.append(jnp.where(valid, s, neg))
            fl_chunks.append(jnp.where(valid, beam * vocab + word, _BIG))
        tops_vec = jnp.zeros((16,), jnp.float32)
        flat_vec = jnp.zeros((16,), jnp.int32)
        for t in range(k):
            m = jnp.maximum(jnp.max(sc_chunks[0]), jnp.max(sc_chunks[1]))
            fc0 = jnp.where(sc_chunks[0] == m, fl_chunks[0], _BIG)
            fc1 = jnp.where(sc_chunks[1] == m, fl_chunks[1], _BIG)
            am = jnp.minimum(jnp.min(fc0), jnp.min(fc1))
            sc_chunks[0] = jnp.where(fc0 == am, neg, sc_chunks[0])
            sc_chunks[1] = jnp.where(fc1 == am, neg, sc_chunks[1])
            tops_vec = jnp.where(lane == t, m, tops_vec)
            flat_vec = jnp.where(lane == t, am, flat_vec)
        beam_vec = jnp.zeros((16,), jnp.int32)
        for g in range(1, k):
            beam_vec = beam_vec + jnp.where(flat_vec >= g * vocab, 1, 0)
        word_vec = flat_vec - beam_vec * vocab
        gidx_v[...] = jnp.where(lane < k, b_glob * k + beam_vec, 0)
        pltpu.async_copy(tbo_hbm.at[gidx_v], rows_v, sem).wait()
        pltpu.sync_copy(rows_v.at[pl.ds(0, 8)], rows_hbm.at[pl.ds(b_glob * 8, 8)])
        tops_v[bi, pl.ds(0, 16)] = tops_vec
        words_v[bi, pl.ds(0, 16)] = word_vec
    pltpu.sync_copy(tops_v, tops_hbm.at[pl.ds(wid * 8, 8)])
    pltpu.sync_copy(words_v, words_hbm.at[pl.ds(wid * 8, 8)])


def kernel(logits, cur_scores, cur_size, eos_mask, top_beam_outputs, beam_width):
    Bb, k, seq_len = top_beam_outputs.shape
    rows, vocab = logits.shape
    rb = _ROW_BLOCK
    grid = rows // rb

    cur2 = cur_scores.reshape(rows, 1)
    size_f = cur_size.astype(jnp.float32).reshape(rows, 1)
    eos_f = eos_mask.astype(jnp.float32).reshape(rows, 1)

    vals, idxs, stats = pl.pallas_call(
        functools.partial(_rowtopk_body, k, rb, vocab),
        grid=(grid,),
        in_specs=[
            pl.BlockSpec((rb, vocab), lambda i: (i, 0)),
            pl.BlockSpec((rb, 1), lambda i: (i, 0)),
            pl.BlockSpec((rb, 1), lambda i: (i, 0)),
            pl.BlockSpec((rb, 1), lambda i: (i, 0)),
        ],
        out_specs=[
            pl.BlockSpec((rb, 8), lambda i: (i, 0)),
            pl.BlockSpec((rb, 8), lambda i: (i, 0)),
            pl.BlockSpec((rb, 8), lambda i: (i, 0)),
        ],
        out_shape=[
            jax.ShapeDtypeStruct((rows, 8), jnp.float32),
            jax.ShapeDtypeStruct((rows, 8), jnp.int32),
            jax.ShapeDtypeStruct((rows, 8), jnp.float32),
        ],
    )(logits, cur2, size_f, eos_f)

    nw = Bb // 2
    tbo2 = top_beam_outputs.reshape(Bb * k, seq_len).astype(jnp.int32)

    mesh = plsc.VectorSubcoreMesh(core_axis_name="c", subcore_axis_name="s")
    sc_merge = functools.partial(
        pl.kernel,
        mesh=mesh,
        compiler_params=pltpu.CompilerParams(
            needs_layout_passes=False, use_tc_tiling_on_sc=False),
        out_type=[
            jax.ShapeDtypeStruct((nw * 8, 16), jnp.float32),
            jax.ShapeDtypeStruct((nw * 8, 16), jnp.int32),
            jax.ShapeDtypeStruct((Bb * 8, seq_len), jnp.int32),
        ],
        scratch_types=[
            pltpu.VMEM((rows, 8), jnp.float32),
            pltpu.VMEM((rows, 8), jnp.int32),
            pltpu.VMEM((rows, 8), jnp.float32),
            pltpu.VMEM((16, seq_len), jnp.int32),
            pltpu.VMEM((16,), jnp.int32),
            pltpu.VMEM((8, 16), jnp.float32),
            pltpu.VMEM((8, 16), jnp.int32),
            pltpu.SemaphoreType.DMA,
        ],
    )(functools.partial(_sc_merge_body, k, vocab))
    tops_pad, words_pad, rows_out = sc_merge(vals, idxs, stats, tbo2)

    tops = tops_pad.reshape(nw, 8, 16)[:, :2, :k].reshape(Bb, k)
    words = words_pad.reshape(nw, 8, 16)[:, :2, :k].reshape(Bb, k)
    top_scores = tops + 0.0 * beam_width
    new_outputs = jnp.concatenate(
        [rows_out.reshape(Bb, 8, seq_len)[:, :k, :],
         words.astype(top_beam_outputs.dtype)[:, :, None]],
        axis=-1)
    return top_scores, new_outputs
